# baseline (speedup 1.0000x reference)
import functools

import jax
import jax.numpy as jnp
from jax import lax
from jax.experimental import pallas as pl
from jax.experimental.pallas import tpu as pltpu
from jax.experimental.pallas import tpu_sc as plsc

N = 10000
E = 320000
D = 128

NC = 2
NS = 16
NW = NC * NS
PER_TILE = E // NW
CH = 80
N_CHUNK = PER_TILE // CH
NP = 10112
ROWS_PER_TILE = NP // NS

_MESH = plsc.VectorSubcoreMesh(core_axis_name="c", subcore_axis_name="s")

L = 16
NVEC = PER_TILE // L
NPVEC = NP // L


@functools.partial(
    pl.kernel,
    out_type=jax.ShapeDtypeStruct((NW, NP), jnp.float32),
    mesh=_MESH,
    compiler_params=pltpu.CompilerParams(needs_layout_passes=False),
    scratch_types=[
        pltpu.VMEM((PER_TILE,), jnp.int32),
        pltpu.VMEM((NP,), jnp.float32),
    ],
)
def _sc_degree(dst_hbm, out_hbm, dst_b, hist):
    c = lax.axis_index("c")
    s = lax.axis_index("s")
    wid = s * NC + c
    pltpu.sync_copy(dst_hbm.at[pl.ds(wid * PER_TILE, PER_TILE)], dst_b)
    z16 = jnp.zeros((L,), jnp.float32)
    one16 = jnp.ones((L,), jnp.float32)

    def zbody(i, carry):
        hist[pl.ds(i * L, L)] = z16
        return carry

    lax.fori_loop(0, NPVEC, zbody, 0)

    def body(i, carry):
        idx = dst_b[pl.ds(i * L, L)]
        plsc.addupdate_scatter(hist, [idx], one16)
        return carry

    lax.fori_loop(0, NVEC, body, 0)
    pltpu.sync_copy(hist, out_hbm.at[wid])


@functools.partial(
    pl.kernel,
    out_type=[
        jax.ShapeDtypeStruct((NC, NP, D), jnp.float32),
        jax.ShapeDtypeStruct((NC, NP, D), jnp.float32),
    ],
    mesh=_MESH,
    scratch_types=[
        pltpu.VMEM((PER_TILE,), jnp.int32),
        pltpu.VMEM((N_CHUNK, CH), jnp.int32),
        pltpu.VMEM((2, CH, D), jnp.float32),
        pltpu.VMEM_SHARED((NP, D), jnp.float32),
        pltpu.SemaphoreType.DMA,
        pltpu.SemaphoreType.DMA,
        pltpu.SemaphoreType.DMA,
        pltpu.SemaphoreType.DMA,
    ],
)
def _sc_edge_pass(z1_hbm, z2_hbm, src_hbm, dst_hbm, zeros_hbm,
                  out1_hbm, out2_hbm, src_v, dst_v, rows_v, acc_s,
                  semA, semB, semSA, semSB):
    c = lax.axis_index("c")
    s = lax.axis_index("s")
    wid = s * NC + c
    rows = pl.ds(s * ROWS_PER_TILE, ROWS_PER_TILE)
    pltpu.sync_copy(zeros_hbm, acc_s.at[rows])
    pltpu.sync_copy(src_hbm.at[pl.ds(wid * PER_TILE, PER_TILE)], src_v)
    pltpu.sync_copy(dst_hbm.at[wid], dst_v)
    plsc.subcore_barrier()

    def one_pass(z_hbm, out_hbm):
        def gather_start(i, p, sem):
            pltpu.async_copy(z_hbm.at[src_v.at[pl.ds(i * CH, CH)]],
                             rows_v.at[p], sem)

        def gather_wait(i, p, sem):
            pltpu.make_async_copy(
                z_hbm.at[src_v.at[pl.ds(i * CH, CH)]], rows_v.at[p], sem).wait()

        def scatter_start(i, p, sem):
            pltpu.async_copy(rows_v.at[p], acc_s.at[dst_v.at[i]], sem,
                             add=True)

        def scatter_wait(i, p, sem):
            pltpu.make_async_copy(rows_v.at[p], acc_s.at[dst_v.at[i]],
                                  sem).wait()

        gather_start(0, 0, semA)

        def body(i, carry):
            @pl.when(i % 2 == 0)
            def _():
                @pl.when(i >= 1)
                def _():
                    scatter_wait(i - 1, 1, semSB)

                gather_start(i + 1, 1, semB)
                gather_wait(i, 0, semA)
                scatter_start(i, 0, semSA)

            @pl.when(i % 2 == 1)
            def _():
                scatter_wait(i - 1, 0, semSA)
                gather_start(i + 1, 0, semA)
                gather_wait(i, 1, semB)
                scatter_start(i, 1, semSB)

            return carry

        lax.fori_loop(0, N_CHUNK - 1, body, 0)
        scatter_wait(N_CHUNK - 2, 1, semSB)
        gather_wait(N_CHUNK - 1, 0, semA)
        scatter_start(N_CHUNK - 1, 0, semSA)
        scatter_wait(N_CHUNK - 1, 0, semSA)
        plsc.subcore_barrier()
        pltpu.sync_copy(acc_s.at[rows], out_hbm.at[c, rows])

    one_pass(z1_hbm, out1_hbm)
    pltpu.sync_copy(zeros_hbm, acc_s.at[rows])
    plsc.subcore_barrier()
    one_pass(z2_hbm, out2_hbm)


LCAP = PER_TILE + 2 * L
TRASH = LCAP - 1


@functools.partial(
    pl.kernel,
    out_type=jax.ShapeDtypeStruct((NC, NS, 8, D), jnp.float32),
    mesh=_MESH,
    compiler_params=pltpu.CompilerParams(needs_layout_passes=False),
    scratch_types=[
        pltpu.VMEM((PER_TILE,), jnp.int32),
        pltpu.VMEM((PER_TILE,), jnp.int32),
        pltpu.VMEM((LCAP,), jnp.int32),
        pltpu.VMEM((LCAP,), jnp.int32),
        pltpu.VMEM((2, L), jnp.int32),
        pltpu.VMEM((2,), jnp.int32),
        pltpu.VMEM((L, D), jnp.float32),
        pltpu.VMEM((L, D), jnp.float32),
        pltpu.VMEM((8, D), jnp.float32),
        pltpu.SemaphoreType.DMA,
    ],
)
def _sc_filter(src_hbm, dst_hbm, ops_hbm, ops2_hbm, q1_hbm, q2_hbm, out_hbm,
               dst_b, src_b, list1, list2, ops_v, idx2_v, rows1, rows2,
               uacc, sem):
    c = lax.axis_index("c")
    s = lax.axis_index("s")
    wid = s * NC + c
    base = wid * PER_TILE
    pltpu.sync_copy(dst_hbm.at[pl.ds(base, PER_TILE)], dst_b)
    pltpu.sync_copy(src_hbm.at[pl.ds(base, PER_TILE)], src_b)
    pltpu.sync_copy(ops_hbm, ops_v)
    op1v = ops_v[0, :]
    op2v = ops_v[1, :]
    z16 = jnp.zeros((L,), jnp.float32)
    for r in range(8):
        for j in range(D // L):
            uacc[r, pl.ds(j * L, L)] = z16

    def process_vreg(i, c1, c2):
        dv = dst_b[pl.ds(i * L, L)]
        sv = src_b[pl.ds(i * L, L)]
        m1 = dv == op1v
        m2 = dv == op2v
        one16 = jnp.ones((L,), jnp.int32)
        trash16 = jnp.full((L,), TRASH, jnp.int32)
        pos1 = plsc.cumsum(m1.astype(jnp.int32))
        pos2 = plsc.cumsum(m2.astype(jnp.int32))
        c1v = jnp.full((L,), c1, jnp.int32)
        c2v = jnp.full((L,), c2, jnp.int32)
        idx1 = jnp.where(m1, c1v + pos1 - one16, trash16)
        idx2 = jnp.where(m2, c2v + pos2 - one16, trash16)
        plsc.store_scatter(list1, [idx1], sv)
        plsc.store_scatter(list2, [idx2], sv)
        return c1 + jnp.max(pos1), c2 + jnp.max(pos2)

    VPC = 25

    def scan_chunk(ci, carry):
        anym = jnp.zeros((L,), jnp.int32) > jnp.zeros((L,), jnp.int32)
        for j in range(VPC):
            dv = dst_b[pl.ds((ci * VPC + j) * L, L)]
            anym = anym | (dv == op1v) | (dv == op2v)
        pc = plsc.all_reduce_population_count(anym)

        def slow(c1, c2):
            return lax.fori_loop(
                ci * VPC, (ci + 1) * VPC,
                lambda j, cc: process_vreg(j, cc[0], cc[1]), (c1, c2))

        def fast(c1, c2):
            return c1, c2

        return lax.cond(pc[0] > 0, slow, fast, *carry)

    cnt1, cnt2 = lax.fori_loop(0, NVEC // VPC, scan_chunk,
                               (jnp.int32(0), jnp.int32(0)))

    padn16 = jnp.full((L,), N, jnp.int32)

    def accumulate(lst, cnt, r1, r2):
        lst[pl.ds(cnt, L)] = padn16
        nch = (cnt + (L - 1)) // L

        def body(k, carry):
            ids = lst.at[pl.ds(k * L, L)]
            pltpu.async_copy(q1_hbm.at[ids], rows1, sem).wait()
            pltpu.async_copy(q2_hbm.at[ids], rows2, sem).wait()

            def row_add(t, cc):
                for j in range(D // L):
                    sl = pl.ds(j * L, L)
                    uacc[r1, sl] = uacc[r1, sl] + rows1[t, sl]
                    uacc[r2, sl] = uacc[r2, sl] + rows2[t, sl]
                return cc

            lax.fori_loop(0, L, row_add, 0)
            return carry

        lax.fori_loop(0, nch, body, 0)

    accumulate(list1, cnt1, 0, 2)
    accumulate(list2, cnt2, 1, 3)

    @pl.when(wid == 0)
    def _():
        pltpu.sync_copy(ops2_hbm, idx2_v)
        pltpu.async_copy(q1_hbm.at[idx2_v], rows1.at[pl.ds(0, 2)], sem).wait()
        pltpu.async_copy(q2_hbm.at[idx2_v], rows2.at[pl.ds(0, 2)], sem).wait()
        for j in range(D // L):
            sl = pl.ds(j * L, L)
            uacc[4, sl] = rows1[0, sl]
            uacc[5, sl] = rows1[1, sl]
            uacc[6, sl] = rows2[0, sl]
            uacc[7, sl] = rows2[1, sl]

    pltpu.sync_copy(uacc, out_hbm.at[c, s])


ROW_BLK = 400
N_BLK = N // ROW_BLK


def _tc_pre(x, deg_part, W1, W2):
    def body(x_ref, p_ref, w1_ref, w2_ref, o1_ref, o2_ref, d_ref):
        deg = jnp.sum(p_ref[...], axis=0) + 1.0
        d = lax.rsqrt(deg)
        d_ref[...] = d
        xd = d * x_ref[...]
        o1_ref[...] = jnp.dot(xd, w1_ref[...],
                              preferred_element_type=jnp.float32)
        o2_ref[...] = jnp.dot(xd, w2_ref[...],
                              preferred_element_type=jnp.float32)

    return pl.pallas_call(
        body,
        grid=(N_BLK,),
        in_specs=[
            pl.BlockSpec((ROW_BLK, D), lambda i: (i, 0)),
            pl.BlockSpec((NW, ROW_BLK, 1), lambda i: (0, i, 0)),
            pl.BlockSpec((D, D), lambda i: (0, 0)),
            pl.BlockSpec((D, D), lambda i: (0, 0)),
        ],
        out_specs=[
            pl.BlockSpec((ROW_BLK, D), lambda i: (i, 0)),
            pl.BlockSpec((ROW_BLK, D), lambda i: (i, 0)),
            pl.BlockSpec((ROW_BLK, 1), lambda i: (i, 0)),
        ],
        out_shape=[
            jax.ShapeDtypeStruct((N, D), jnp.float32),
            jax.ShapeDtypeStruct((N, D), jnp.float32),
            jax.ShapeDtypeStruct((N, 1), jnp.float32),
        ],
    )(x, deg_part, W1, W2)


NQ = N + ROW_BLK


def _tc_q(p1, p2, z1, z2, dinv, b1, b2):
    def body(p1_ref, p2_ref, z1_ref, z2_ref, d_ref, b1_ref, b2_ref,
             o1_ref, o2_ref):
        k = pl.program_id(0)
        d = d_ref[...]
        h1 = jnp.maximum(d * (p1_ref[0] + p1_ref[1] + z1_ref[...])
                         + b1_ref[...], 0.0)
        h2 = jnp.maximum(d * (p2_ref[0] + p2_ref[1] + z2_ref[...])
                         + b2_ref[...], 0.0)
        live = (k < N_BLK).astype(jnp.float32)
        o1_ref[...] = live * d * h1
        o2_ref[...] = live * d * h2

    clamp = lambda i: (jnp.minimum(i, N_BLK - 1), 0)
    clamp3 = lambda i: (0, jnp.minimum(i, N_BLK - 1), 0)
    return pl.pallas_call(
        body,
        grid=(N_BLK + 1,),
        in_specs=[
            pl.BlockSpec((NC, ROW_BLK, D), clamp3),
            pl.BlockSpec((NC, ROW_BLK, D), clamp3),
            pl.BlockSpec((ROW_BLK, D), clamp),
            pl.BlockSpec((ROW_BLK, D), clamp),
            pl.BlockSpec((ROW_BLK, 1), clamp),
            pl.BlockSpec((1, D), lambda i: (0, 0)),
            pl.BlockSpec((1, D), lambda i: (0, 0)),
        ],
        out_specs=[
            pl.BlockSpec((ROW_BLK, D), lambda i: (i, 0)),
            pl.BlockSpec((ROW_BLK, D), lambda i: (i, 0)),
        ],
        out_shape=[
            jax.ShapeDtypeStruct((NQ, D), jnp.float32),
            jax.ShapeDtypeStruct((NQ, D), jnp.float32),
        ],
    )(p1, p2, z1, z2, dinv, b1, b2)


def _tc_finish(slabs, dinv, W1b, b1b, W2b, b2b, ops):
    def body(ops_ref, sl_ref, d_ref, w1_ref, b1_ref, w2_ref, b2_ref, o_ref):
        u = jnp.sum(sl_ref[...], axis=0)
        o1 = ops_ref[0]
        o2 = ops_ref[1]
        u1 = u[0:2] + u[4:6]
        u2 = u[2:4] + u[6:8]
        d1 = d_ref[pl.ds(o1, 1), :]
        d2 = d_ref[pl.ds(o2, 1), :]
        a = jnp.dot(u1, w1_ref[...], preferred_element_type=jnp.float32)
        b = jnp.dot(u2, w2_ref[...], preferred_element_type=jnp.float32)
        y11 = d1 * a[0:1] + b1_ref[...]
        y12 = d2 * a[1:2] + b1_ref[...]
        y21 = d1 * b[0:1] + b2_ref[...]
        y22 = d2 * b[1:2] + b2_ref[...]
        o_ref[0] = jnp.sum(y11 * y12)
        o_ref[1] = jnp.sum(y21 * y22)

    return pl.pallas_call(
        body,
        in_specs=[
            pl.BlockSpec(memory_space=pltpu.SMEM),
            pl.BlockSpec(memory_space=pltpu.VMEM),
            pl.BlockSpec(memory_space=pltpu.VMEM),
            pl.BlockSpec(memory_space=pltpu.VMEM),
            pl.BlockSpec(memory_space=pltpu.VMEM),
            pl.BlockSpec(memory_space=pltpu.VMEM),
            pl.BlockSpec(memory_space=pltpu.VMEM),
        ],
        out_specs=pl.BlockSpec(memory_space=pltpu.SMEM),
        out_shape=jax.ShapeDtypeStruct((2,), jnp.float32),
    )(ops, slabs, dinv, W1b, b1b, W2b, b2b)


def kernel(x, edge_index, op1, op2, W1a, b1a, W1b, b1b, W2a, b2a, W2b, b2b):
    src = edge_index[0]
    dst = edge_index[1]
    zeros_row = jnp.zeros((ROWS_PER_TILE, D), jnp.float32)
    ops = jnp.stack([op1, op2]).astype(jnp.int32)
    ops16 = jnp.broadcast_to(ops[:, None], (2, L))

    deg_part = _sc_degree(dst)
    dst_t = dst.reshape(NW, N_CHUNK, CH)
    z1, z2, dinv = _tc_pre(x, deg_part.reshape(NW, NP, 1), W1a, W2a)
    p1, p2 = _sc_edge_pass(z1, z2, src, dst_t, zeros_row)
    q1, q2 = _tc_q(p1, p2, z1, z2, dinv, b1a.reshape(1, D), b2a.reshape(1, D))

    slabs = _sc_filter(src, dst, ops16, ops, q1, q2)
    slabs = slabs.reshape(NC * NS, 8, D)
    return _tc_finish(slabs, dinv, W1b, b1b.reshape(1, D),
                      W2b, b2b.reshape(1, D), ops)


# speedup vs baseline: 1.0008x; 1.0008x over previous
import functools

import jax
import jax.numpy as jnp
from jax import lax
from jax.experimental import pallas as pl
from jax.experimental.pallas import tpu as pltpu
from jax.experimental.pallas import tpu_sc as plsc

N = 10000
E = 320000
D = 128

NC = 2
NS = 16
NW = NC * NS
PER_TILE = E // NW
CH = 80
N_CHUNK = PER_TILE // CH
NP = 10112
ROWS_PER_TILE = NP // NS

_MESH = plsc.VectorSubcoreMesh(core_axis_name="c", subcore_axis_name="s")

L = 16
NVEC = PER_TILE // L
NPVEC = NP // L


@functools.partial(
    pl.kernel,
    out_type=jax.ShapeDtypeStruct((NW, NP), jnp.float32),
    mesh=_MESH,
    compiler_params=pltpu.CompilerParams(needs_layout_passes=False),
    scratch_types=[
        pltpu.VMEM((PER_TILE,), jnp.int32),
        pltpu.VMEM((NP,), jnp.float32),
    ],
)
def _sc_degree(dst_hbm, out_hbm, dst_b, hist):
    c = lax.axis_index("c")
    s = lax.axis_index("s")
    wid = s * NC + c
    pltpu.sync_copy(dst_hbm.at[pl.ds(wid * PER_TILE, PER_TILE)], dst_b)
    z16 = jnp.zeros((L,), jnp.float32)
    one16 = jnp.ones((L,), jnp.float32)

    def zbody(i, carry):
        hist[pl.ds(i * L, L)] = z16
        return carry

    lax.fori_loop(0, NPVEC, zbody, 0)

    def body(i, carry):
        idx = dst_b[pl.ds(i * L, L)]
        plsc.addupdate_scatter(hist, [idx], one16)
        return carry

    lax.fori_loop(0, NVEC, body, 0)
    pltpu.sync_copy(hist, out_hbm.at[wid])


@functools.partial(
    pl.kernel,
    out_type=[
        jax.ShapeDtypeStruct((NC, NP, D), jnp.float32),
        jax.ShapeDtypeStruct((NC, NP, D), jnp.float32),
    ],
    mesh=_MESH,
    scratch_types=[
        pltpu.VMEM((PER_TILE,), jnp.int32),
        pltpu.VMEM((N_CHUNK, CH), jnp.int32),
        pltpu.VMEM((2, CH, D), jnp.float32),
        pltpu.VMEM_SHARED((NP, D), jnp.float32),
        pltpu.SemaphoreType.DMA,
        pltpu.SemaphoreType.DMA,
    ],
)
def _sc_edge_pass(z1_hbm, z2_hbm, src_hbm, dst_hbm, zeros_hbm,
                  out1_hbm, out2_hbm, src_v, dst_v, rows_v, acc_s, semA, semB):
    c = lax.axis_index("c")
    s = lax.axis_index("s")
    wid = s * NC + c
    rows = pl.ds(s * ROWS_PER_TILE, ROWS_PER_TILE)
    pltpu.sync_copy(zeros_hbm, acc_s.at[rows])
    pltpu.sync_copy(src_hbm.at[pl.ds(wid * PER_TILE, PER_TILE)], src_v)
    pltpu.sync_copy(dst_hbm.at[wid], dst_v)
    plsc.subcore_barrier()

    def one_pass(z_hbm, out_hbm):
        def gather_start(i, p, sem):
            pltpu.async_copy(z_hbm.at[src_v.at[pl.ds(i * CH, CH)]],
                             rows_v.at[p], sem)

        def gather_wait(i, p, sem):
            pltpu.make_async_copy(
                z_hbm.at[src_v.at[pl.ds(i * CH, CH)]], rows_v.at[p], sem).wait()

        def scatter(i, p):
            pltpu.sync_copy(rows_v.at[p], acc_s.at[dst_v.at[i]], add=True)

        gather_start(0, 0, semA)

        def body(i, carry):
            @pl.when(i % 2 == 0)
            def _():
                gather_start(i + 1, 1, semB)
                gather_wait(i, 0, semA)
                scatter(i, 0)

            @pl.when(i % 2 == 1)
            def _():
                gather_start(i + 1, 0, semA)
                gather_wait(i, 1, semB)
                scatter(i, 1)

            return carry

        lax.fori_loop(0, N_CHUNK - 1, body, 0)
        gather_wait(N_CHUNK - 1, 0, semA)
        scatter(N_CHUNK - 1, 0)
        plsc.subcore_barrier()
        pltpu.sync_copy(acc_s.at[rows], out_hbm.at[c, rows])

    one_pass(z1_hbm, out1_hbm)
    pltpu.sync_copy(zeros_hbm, acc_s.at[rows])
    plsc.subcore_barrier()
    one_pass(z2_hbm, out2_hbm)


LCAP = PER_TILE + 2 * L
TRASH = LCAP - 1


@functools.partial(
    pl.kernel,
    out_type=jax.ShapeDtypeStruct((NC, NS, 8, D), jnp.float32),
    mesh=_MESH,
    compiler_params=pltpu.CompilerParams(needs_layout_passes=False),
    scratch_types=[
        pltpu.VMEM((PER_TILE,), jnp.int32),
        pltpu.VMEM((PER_TILE,), jnp.int32),
        pltpu.VMEM((LCAP,), jnp.int32),
        pltpu.VMEM((LCAP,), jnp.int32),
        pltpu.VMEM((2, L), jnp.int32),
        pltpu.VMEM((2,), jnp.int32),
        pltpu.VMEM((L, D), jnp.float32),
        pltpu.VMEM((L, D), jnp.float32),
        pltpu.VMEM((8, D), jnp.float32),
        pltpu.SemaphoreType.DMA,
    ],
)
def _sc_filter(src_hbm, dst_hbm, ops_hbm, ops2_hbm, q1_hbm, q2_hbm, out_hbm,
               dst_b, src_b, list1, list2, ops_v, idx2_v, rows1, rows2,
               uacc, sem):
    c = lax.axis_index("c")
    s = lax.axis_index("s")
    wid = s * NC + c
    base = wid * PER_TILE
    pltpu.sync_copy(dst_hbm.at[pl.ds(base, PER_TILE)], dst_b)
    pltpu.sync_copy(src_hbm.at[pl.ds(base, PER_TILE)], src_b)
    pltpu.sync_copy(ops_hbm, ops_v)
    op1v = ops_v[0, :]
    op2v = ops_v[1, :]
    z16 = jnp.zeros((L,), jnp.float32)
    for r in range(8):
        for j in range(D // L):
            uacc[r, pl.ds(j * L, L)] = z16

    def process_vreg(i, c1, c2):
        dv = dst_b[pl.ds(i * L, L)]
        sv = src_b[pl.ds(i * L, L)]
        m1 = dv == op1v
        m2 = dv == op2v
        one16 = jnp.ones((L,), jnp.int32)
        trash16 = jnp.full((L,), TRASH, jnp.int32)
        pos1 = plsc.cumsum(m1.astype(jnp.int32))
        pos2 = plsc.cumsum(m2.astype(jnp.int32))
        c1v = jnp.full((L,), c1, jnp.int32)
        c2v = jnp.full((L,), c2, jnp.int32)
        idx1 = jnp.where(m1, c1v + pos1 - one16, trash16)
        idx2 = jnp.where(m2, c2v + pos2 - one16, trash16)
        plsc.store_scatter(list1, [idx1], sv)
        plsc.store_scatter(list2, [idx2], sv)
        return c1 + jnp.max(pos1), c2 + jnp.max(pos2)

    VPC = 25

    def scan_chunk(ci, carry):
        anym = jnp.zeros((L,), jnp.int32) > jnp.zeros((L,), jnp.int32)
        for j in range(VPC):
            dv = dst_b[pl.ds((ci * VPC + j) * L, L)]
            anym = anym | (dv == op1v) | (dv == op2v)
        pc = plsc.all_reduce_population_count(anym)

        def slow(c1, c2):
            return lax.fori_loop(
                ci * VPC, (ci + 1) * VPC,
                lambda j, cc: process_vreg(j, cc[0], cc[1]), (c1, c2))

        def fast(c1, c2):
            return c1, c2

        return lax.cond(pc[0] > 0, slow, fast, *carry)

    cnt1, cnt2 = lax.fori_loop(0, NVEC // VPC, scan_chunk,
                               (jnp.int32(0), jnp.int32(0)))

    padn16 = jnp.full((L,), N, jnp.int32)

    def accumulate(lst, cnt, r1, r2):
        lst[pl.ds(cnt, L)] = padn16
        nch = (cnt + (L - 1)) // L

        def body(k, carry):
            ids = lst.at[pl.ds(k * L, L)]
            pltpu.async_copy(q1_hbm.at[ids], rows1, sem).wait()
            pltpu.async_copy(q2_hbm.at[ids], rows2, sem).wait()

            def row_add(t, cc):
                for j in range(D // L):
                    sl = pl.ds(j * L, L)
                    uacc[r1, sl] = uacc[r1, sl] + rows1[t, sl]
                    uacc[r2, sl] = uacc[r2, sl] + rows2[t, sl]
                return cc

            lax.fori_loop(0, L, row_add, 0)
            return carry

        lax.fori_loop(0, nch, body, 0)

    accumulate(list1, cnt1, 0, 2)
    accumulate(list2, cnt2, 1, 3)

    @pl.when(wid == 0)
    def _():
        pltpu.sync_copy(ops2_hbm, idx2_v)
        pltpu.async_copy(q1_hbm.at[idx2_v], rows1.at[pl.ds(0, 2)], sem).wait()
        pltpu.async_copy(q2_hbm.at[idx2_v], rows2.at[pl.ds(0, 2)], sem).wait()
        for j in range(D // L):
            sl = pl.ds(j * L, L)
            uacc[4, sl] = rows1[0, sl]
            uacc[5, sl] = rows1[1, sl]
            uacc[6, sl] = rows2[0, sl]
            uacc[7, sl] = rows2[1, sl]

    pltpu.sync_copy(uacc, out_hbm.at[c, s])


ROW_BLK = 400
N_BLK = N // ROW_BLK


def _tc_pre(x, deg_part, W1, W2):
    def body(x_ref, p_ref, w1_ref, w2_ref, o1_ref, o2_ref, d_ref):
        deg = jnp.sum(p_ref[...], axis=0) + 1.0
        d = lax.rsqrt(deg)
        d_ref[...] = d
        xd = d * x_ref[...]
        o1_ref[...] = jnp.dot(xd, w1_ref[...],
                              preferred_element_type=jnp.float32)
        o2_ref[...] = jnp.dot(xd, w2_ref[...],
                              preferred_element_type=jnp.float32)

    return pl.pallas_call(
        body,
        grid=(N_BLK,),
        in_specs=[
            pl.BlockSpec((ROW_BLK, D), lambda i: (i, 0)),
            pl.BlockSpec((NW, ROW_BLK, 1), lambda i: (0, i, 0)),
            pl.BlockSpec((D, D), lambda i: (0, 0)),
            pl.BlockSpec((D, D), lambda i: (0, 0)),
        ],
        out_specs=[
            pl.BlockSpec((ROW_BLK, D), lambda i: (i, 0)),
            pl.BlockSpec((ROW_BLK, D), lambda i: (i, 0)),
            pl.BlockSpec((ROW_BLK, 1), lambda i: (i, 0)),
        ],
        out_shape=[
            jax.ShapeDtypeStruct((N, D), jnp.float32),
            jax.ShapeDtypeStruct((N, D), jnp.float32),
            jax.ShapeDtypeStruct((N, 1), jnp.float32),
        ],
    )(x, deg_part, W1, W2)


NQ = N + ROW_BLK


def _tc_q(p1, p2, z1, z2, dinv, b1, b2):
    def body(p1_ref, p2_ref, z1_ref, z2_ref, d_ref, b1_ref, b2_ref,
             o1_ref, o2_ref):
        k = pl.program_id(0)
        d = d_ref[...]
        h1 = jnp.maximum(d * (p1_ref[0] + p1_ref[1] + z1_ref[...])
                         + b1_ref[...], 0.0)
        h2 = jnp.maximum(d * (p2_ref[0] + p2_ref[1] + z2_ref[...])
                         + b2_ref[...], 0.0)
        live = (k < N_BLK).astype(jnp.float32)
        o1_ref[...] = live * d * h1
        o2_ref[...] = live * d * h2

    clamp = lambda i: (jnp.minimum(i, N_BLK - 1), 0)
    clamp3 = lambda i: (0, jnp.minimum(i, N_BLK - 1), 0)
    return pl.pallas_call(
        body,
        grid=(N_BLK + 1,),
        in_specs=[
            pl.BlockSpec((NC, ROW_BLK, D), clamp3),
            pl.BlockSpec((NC, ROW_BLK, D), clamp3),
            pl.BlockSpec((ROW_BLK, D), clamp),
            pl.BlockSpec((ROW_BLK, D), clamp),
            pl.BlockSpec((ROW_BLK, 1), clamp),
            pl.BlockSpec((1, D), lambda i: (0, 0)),
            pl.BlockSpec((1, D), lambda i: (0, 0)),
        ],
        out_specs=[
            pl.BlockSpec((ROW_BLK, D), lambda i: (i, 0)),
            pl.BlockSpec((ROW_BLK, D), lambda i: (i, 0)),
        ],
        out_shape=[
            jax.ShapeDtypeStruct((NQ, D), jnp.float32),
            jax.ShapeDtypeStruct((NQ, D), jnp.float32),
        ],
    )(p1, p2, z1, z2, dinv, b1, b2)


def _tc_finish(slabs, dinv, W1b, b1b, W2b, b2b, ops):
    def body(ops_ref, sl_ref, d_ref, w1_ref, b1_ref, w2_ref, b2_ref, o_ref):
        u = jnp.sum(sl_ref[...], axis=0)
        o1 = ops_ref[0]
        o2 = ops_ref[1]
        u1 = u[0:2] + u[4:6]
        u2 = u[2:4] + u[6:8]
        d1 = d_ref[pl.ds(o1, 1), :]
        d2 = d_ref[pl.ds(o2, 1), :]
        a = jnp.dot(u1, w1_ref[...], preferred_element_type=jnp.float32)
        b = jnp.dot(u2, w2_ref[...], preferred_element_type=jnp.float32)
        y11 = d1 * a[0:1] + b1_ref[...]
        y12 = d2 * a[1:2] + b1_ref[...]
        y21 = d1 * b[0:1] + b2_ref[...]
        y22 = d2 * b[1:2] + b2_ref[...]
        o_ref[0] = jnp.sum(y11 * y12)
        o_ref[1] = jnp.sum(y21 * y22)

    return pl.pallas_call(
        body,
        in_specs=[
            pl.BlockSpec(memory_space=pltpu.SMEM),
            pl.BlockSpec(memory_space=pltpu.VMEM),
            pl.BlockSpec(memory_space=pltpu.VMEM),
            pl.BlockSpec(memory_space=pltpu.VMEM),
            pl.BlockSpec(memory_space=pltpu.VMEM),
            pl.BlockSpec(memory_space=pltpu.VMEM),
            pl.BlockSpec(memory_space=pltpu.VMEM),
        ],
        out_specs=pl.BlockSpec(memory_space=pltpu.SMEM),
        out_shape=jax.ShapeDtypeStruct((2,), jnp.float32),
    )(ops, slabs, dinv, W1b, b1b, W2b, b2b)


def kernel(x, edge_index, op1, op2, W1a, b1a, W1b, b1b, W2a, b2a, W2b, b2b):
    src = edge_index[0]
    dst = edge_index[1]
    zeros_row = jnp.zeros((ROWS_PER_TILE, D), jnp.float32)
    ops = jnp.stack([op1, op2]).astype(jnp.int32)
    ops16 = jnp.broadcast_to(ops[:, None], (2, L))

    deg_part = _sc_degree(dst)
    dst_t = dst.reshape(NW, N_CHUNK, CH)
    z1, z2, dinv = _tc_pre(x, deg_part.reshape(NW, NP, 1), W1a, W2a)
    p1, p2 = _sc_edge_pass(z1, z2, src, dst_t, zeros_row)
    q1, q2 = _tc_q(p1, p2, z1, z2, dinv, b1a.reshape(1, D), b2a.reshape(1, D))

    slabs = _sc_filter(src, dst, ops16, ops, q1, q2)
    slabs = slabs.reshape(NC * NS, 8, D)
    return _tc_finish(slabs, dinv, W1b, b1b.reshape(1, D),
                      W2b, b2b.reshape(1, D), ops)
